# Initial kernel scaffold; baseline (speedup 1.0000x reference)
import jax
import jax.numpy as jnp
from jax.experimental import pallas as pl


def kernel(x, weight):
    raise NotImplementedError("write your pallas kernel here")


# speedup vs baseline: 5.6571x; 5.6571x over previous
import functools

import jax
import jax.numpy as jnp
from jax import lax
from jax.experimental import pallas as pl
from jax.experimental.pallas import tpu as pltpu
from jax.experimental.pallas import tpu_sc as plsc

NUM_CORES = 2
NUM_SUBCORES = 16
NW = NUM_CORES * NUM_SUBCORES

CHUNK = 128
NBUF = 8
AHEAD = 4


@functools.partial(jax.jit, static_argnames=("n_rows", "dim"))
def _sc_gather(weight, idx2d, *, n_rows, dim):
    chunks_total = idx2d.shape[0]
    cpw = chunks_total // NW

    mesh = plsc.VectorSubcoreMesh(core_axis_name="c", subcore_axis_name="s")

    @functools.partial(
        pl.kernel,
        mesh=mesh,
        compiler_params=pltpu.CompilerParams(use_tc_tiling_on_sc=False),
        out_type=jax.ShapeDtypeStruct((n_rows, dim), jnp.float32),
        scratch_types=[
            pltpu.VMEM((cpw, CHUNK), jnp.int32),
            pltpu.VMEM((NBUF, CHUNK, dim), jnp.float32),
            pltpu.SemaphoreType.DMA((NBUF,)),
            pltpu.SemaphoreType.DMA((NBUF,)),
        ],
    )
    def k(w_hbm, idx_hbm, out_hbm, idx_v, rows_v, gsem, ssem):
        wid = lax.axis_index("s") * NUM_CORES + lax.axis_index("c")
        base_chunk = wid * cpw
        base_row = wid * cpw * CHUNK

        pltpu.sync_copy(idx_hbm.at[pl.ds(base_chunk, cpw)], idx_v)

        def gather(c, b):
            pltpu.async_copy(w_hbm.at[idx_v.at[c]], rows_v.at[b], gsem.at[b])

        def gather_wait(c, b):
            pltpu.make_async_copy(
                w_hbm.at[idx_v.at[c]], rows_v.at[b], gsem.at[b]).wait()

        def store(c, b):
            pltpu.async_copy(
                rows_v.at[b], out_hbm.at[pl.ds(base_row + c * CHUNK, CHUNK)],
                ssem.at[b])

        def store_wait(c, b):
            pltpu.make_async_copy(
                rows_v.at[b], out_hbm.at[pl.ds(base_row + c * CHUNK, CHUNK)],
                ssem.at[b]).wait()

        for b in range(AHEAD):
            gather(b, b)

        @pl.loop(0, cpw, step=NBUF)
        def _(i0):
            for b in range(NBUF):
                i = i0 + b
                gather_wait(i, b)
                store(i, b)
                b2 = (b + AHEAD) % NBUF

                @pl.when(i >= NBUF - AHEAD)
                def _():
                    store_wait(i + AHEAD - NBUF, b2)

                @pl.when(i + AHEAD < cpw)
                def _():
                    gather(i + AHEAD, b2)

        for b in range(AHEAD):
            c = cpw - AHEAD + b
            store_wait(c, c % NBUF)

    return k(weight, idx2d)


def kernel(x, weight):
    b, f = x.shape
    n_rows = b * f
    dim = weight.shape[1]
    idx2d = x.reshape(n_rows // CHUNK, CHUNK)
    out = _sc_gather(weight, idx2d, n_rows=n_rows, dim=dim)
    return out.reshape(b, f, dim)
